# Initial kernel scaffold; baseline (speedup 1.0000x reference)
import jax
import jax.numpy as jnp
from jax.experimental import pallas as pl


def kernel(x, index, dim_size, W1, b1, W2, b2, Wt, bt):
    raise NotImplementedError("write your pallas kernel here")


# speedup vs baseline: 4.3733x; 4.3733x over previous
import jax
import jax.numpy as jnp
from jax import lax
from jax.experimental import pallas as pl
from jax.experimental.pallas import tpu as pltpu
from jax.experimental.pallas import tpu_sc as plsc

N = 100000
D = 128
S = 5000
R1 = 800
G1 = N // R1
B = 128
NB_FULL = N // B
TAIL = N - NB_FULL * B
NW = 32
KPW = (NB_FULL + NW - 1) // NW
ACC_ROWS = 5120

_HI = jax.lax.Precision.HIGHEST


def _k1_body(x_ref, W1_ref, b1_ref, w2_ref, b2_ref, Wt_ref, bt_ref,
             h_ref, g_ref, gmax_ref, m_scr):
    i = pl.program_id(0)
    xb = x_ref[...]
    a1 = jnp.maximum(
        jnp.dot(xb, W1_ref[...], preferred_element_type=jnp.float32,
                precision=_HI) + b1_ref[...], 0.0)
    g = jnp.sum(a1 * w2_ref[...], axis=1) + b2_ref[0, 0]
    h_ref[...] = jnp.maximum(
        jnp.dot(xb, Wt_ref[...], preferred_element_type=jnp.float32,
                precision=_HI) + bt_ref[...], 0.0)
    g_ref[0, 0] = g

    @pl.when(i == 0)
    def _():
        m_scr[0] = -3e38

    m_scr[0] = jnp.maximum(m_scr[0], jnp.max(g))
    gmax_ref[...] = jnp.broadcast_to(m_scr[0], (1, 1))


_GDN = lax.GatherDimensionNumbers(
    offset_dims=(), collapsed_slice_dims=(0,), start_index_map=(0,))


def _bcast_lane(v, r):
    return lax.gather(v, jnp.full((16, 1), r, jnp.int32), _GDN,
                      slice_sizes=(1,),
                      mode=lax.GatherScatterMode.PROMISE_IN_BOUNDS)


def _zero_rows(ref, n_rows):
    zero16 = jnp.zeros((16,), jnp.float32)

    def zrow(r, carry):
        for c in range(D // 16):
            ref[r, pl.ds(c * 16, 16)] = zero16
        return carry

    lax.fori_loop(0, n_rows, zrow, 0)


def _k2a_body(h_hbm, g_hbm, i_hbm, gm_hbm, out_hbm,
              gv, iv, hv, ehv, gmv, zv, gt, it, ht, eht, acc):
    cid = lax.axis_index("c")
    sid = lax.axis_index("s")
    wid = sid * 2 + cid

    _zero_rows(zv, 32)
    for k in range(10):
        pltpu.sync_copy(zv, acc.at[pl.ds(sid * 320 + k * 32, 32)])
    plsc.subcore_barrier()

    pltpu.sync_copy(gm_hbm, gmv)
    gmx = gmv[...]

    def scale_rows(n_rows, gref, href, ehref):
        def grp(j, carry):
            ev = jnp.exp(gref[pl.ds(j * 16, 16)] - gmx)
            for r in range(16):
                row = j * 16 + r
                eb = _bcast_lane(ev, r)
                for c in range(8):
                    ehref[row, pl.ds(c * 16, 16)] = (
                        href[row, pl.ds(c * 16, 16)] * eb)
            return carry

        lax.fori_loop(0, n_rows // 16, grp, 0)

    def blk(k, carry):
        b = k * NW + wid

        @pl.when(b < NB_FULL)
        def _():
            off = b * B
            pltpu.sync_copy(g_hbm.at[pl.ds(off, B)], gv)
            pltpu.sync_copy(i_hbm.at[pl.ds(off, B)], iv)
            pltpu.sync_copy(h_hbm.at[pl.ds(off, B)], hv)
            scale_rows(B, gv, hv, ehv)
            pltpu.sync_copy(ehv, acc.at[iv], add=True)

        return carry

    lax.fori_loop(0, KPW, blk, 0)

    @pl.when(wid == NW - 1)
    def _():
        off = NB_FULL * B
        pltpu.sync_copy(g_hbm.at[pl.ds(off, TAIL)], gt)
        pltpu.sync_copy(i_hbm.at[pl.ds(off, TAIL)], it)
        pltpu.sync_copy(h_hbm.at[pl.ds(off, TAIL)], ht)
        scale_rows(TAIL, gt, ht, eht)
        pltpu.sync_copy(eht, acc.at[it], add=True)

    plsc.subcore_barrier()

    pltpu.sync_copy(acc.at[pl.ds(sid * 320, 320)],
                    out_hbm.at[cid, pl.ds(sid * 320, 320)])


def _k2b_body(g_hbm, i_hbm, gm_hbm, gs_hbm,
              gv, iv, evb, gmv, zv, gt, it, evbt, bnc, gsd, acce):
    cid = lax.axis_index("c")
    sid = lax.axis_index("s")
    wid = sid * 2 + cid
    lane0 = lax.iota(jnp.int32, 16) == 0
    lane = lax.iota(jnp.int32, 16)

    _zero_rows(zv, 32)
    for k in range(10):
        pltpu.sync_copy(zv, acce.at[pl.ds(sid * 320 + k * 32, 32)])
    _zero_rows(evb, B)
    _zero_rows(evbt, TAIL)
    plsc.subcore_barrier()

    pltpu.sync_copy(gm_hbm, gmv)
    gmx = gmv[...]

    def e_rows(n_rows, gref, evref):
        def grp(j, carry):
            ev = jnp.exp(gref[pl.ds(j * 16, 16)] - gmx)
            for r in range(16):
                evref[j * 16 + r, pl.ds(0, 16)] = jnp.where(
                    lane0, _bcast_lane(ev, r), 0.0)
            return carry

        lax.fori_loop(0, n_rows // 16, grp, 0)

    def blk(k, carry):
        b = k * NW + wid

        @pl.when(b < NB_FULL)
        def _():
            off = b * B
            pltpu.sync_copy(g_hbm.at[pl.ds(off, B)], gv)
            pltpu.sync_copy(i_hbm.at[pl.ds(off, B)], iv)
            e_rows(B, gv, evb)
            pltpu.sync_copy(evb, acce.at[iv], add=True)

        return carry

    lax.fori_loop(0, KPW, blk, 0)

    @pl.when(wid == NW - 1)
    def _():
        off = NB_FULL * B
        pltpu.sync_copy(g_hbm.at[pl.ds(off, TAIL)], gt)
        pltpu.sync_copy(i_hbm.at[pl.ds(off, TAIL)], it)
        e_rows(TAIL, gt, evbt)
        pltpu.sync_copy(evbt, acce.at[it], add=True)

    plsc.subcore_barrier()

    for q in range(4):
        pltpu.sync_copy(acce.at[pl.ds(sid * 320 + q * 80, 80)], bnc)
        for j in range(5):
            acc16 = jnp.zeros((16,), jnp.float32)
            for kk in range(16):
                v16 = bnc[j * 16 + kk, pl.ds(0, 16)]
                acc16 = jnp.where(lane == kk, _bcast_lane(v16, 0), acc16)
            gsd[pl.ds(q * 80 + j * 16, 16)] = acc16
    pltpu.sync_copy(gsd, gs_hbm.at[pl.ds(cid * ACC_ROWS + sid * 320, 320)])


def _k3_body(p0_ref, p1_ref, gs_ref, o_ref):
    i = pl.program_id(0)
    num = p0_ref[...] + p1_ref[...]
    den = jnp.sum(gs_ref[:, pl.ds(i * (ACC_ROWS // 5), ACC_ROWS // 5)], axis=0)
    o_ref[...] = num / (den[:, None] + 1e-16)


def kernel(x, index, dim_size, W1, b1, W2, b2, Wt, bt):
    k1 = pl.pallas_call(
        _k1_body,
        grid=(G1,),
        in_specs=[
            pl.BlockSpec((R1, D), lambda i: (i, 0)),
            pl.BlockSpec((D, D // 2), lambda i: (0, 0)),
            pl.BlockSpec((1, D // 2), lambda i: (0, 0)),
            pl.BlockSpec((1, D // 2), lambda i: (0, 0)),
            pl.BlockSpec((1, 1), lambda i: (0, 0)),
            pl.BlockSpec((D, D), lambda i: (0, 0)),
            pl.BlockSpec((1, D), lambda i: (0, 0)),
        ],
        out_specs=[
            pl.BlockSpec((R1, D), lambda i: (i, 0)),
            pl.BlockSpec((1, 1, R1), lambda i: (i, 0, 0)),
            pl.BlockSpec((1, 1), lambda i: (0, 0)),
        ],
        out_shape=[
            jax.ShapeDtypeStruct((N, D), jnp.float32),
            jax.ShapeDtypeStruct((G1, 1, R1), jnp.float32),
            jax.ShapeDtypeStruct((1, 1), jnp.float32),
        ],
        scratch_shapes=[pltpu.SMEM((1,), jnp.float32)],
    )
    h, g3, gmax = k1(x, W1, b1.reshape(1, D // 2), W2.reshape(1, D // 2),
                     b2.reshape(1, 1), Wt, bt.reshape(1, D))
    g_flat = g3.reshape(N)
    gmax16 = jnp.broadcast_to(gmax.reshape(1), (16,))

    mesh = plsc.VectorSubcoreMesh(core_axis_name="c", subcore_axis_name="s",
                                  num_cores=2, num_subcores=16)
    k2a = pl.kernel(
        _k2a_body,
        out_type=jax.ShapeDtypeStruct((2, ACC_ROWS, D), jnp.float32),
        mesh=mesh,
        scratch_types=[
            pltpu.VMEM((B,), jnp.float32),
            pltpu.VMEM((B,), jnp.int32),
            pltpu.VMEM((B, D), jnp.float32),
            pltpu.VMEM((B, D), jnp.float32),
            pltpu.VMEM((16,), jnp.float32),
            pltpu.VMEM((32, D), jnp.float32),
            pltpu.VMEM((TAIL,), jnp.float32),
            pltpu.VMEM((TAIL,), jnp.int32),
            pltpu.VMEM((TAIL, D), jnp.float32),
            pltpu.VMEM((TAIL, D), jnp.float32),
            pltpu.VMEM_SHARED((ACC_ROWS, D), jnp.float32),
        ],
    )
    partial = k2a(h, g_flat, index, gmax16)

    k2b = pl.kernel(
        _k2b_body,
        out_type=jax.ShapeDtypeStruct((2 * ACC_ROWS,), jnp.float32),
        mesh=mesh,
        scratch_types=[
            pltpu.VMEM((B,), jnp.float32),
            pltpu.VMEM((B,), jnp.int32),
            pltpu.VMEM((B, D), jnp.float32),
            pltpu.VMEM((16,), jnp.float32),
            pltpu.VMEM((32, D), jnp.float32),
            pltpu.VMEM((TAIL,), jnp.float32),
            pltpu.VMEM((TAIL,), jnp.int32),
            pltpu.VMEM((TAIL, D), jnp.float32),
            pltpu.VMEM((80, D), jnp.float32),
            pltpu.VMEM((320,), jnp.float32),
            pltpu.VMEM_SHARED((ACC_ROWS, D), jnp.float32),
        ],
    )
    gs = k2b(g_flat, index, gmax16)

    k3 = pl.pallas_call(
        _k3_body,
        grid=(5,),
        in_specs=[
            pl.BlockSpec((ACC_ROWS // 5, D), lambda i: (i, 0)),
            pl.BlockSpec((ACC_ROWS // 5, D), lambda i: (i, 0)),
            pl.BlockSpec((2, ACC_ROWS), lambda i: (0, 0)),
        ],
        out_specs=pl.BlockSpec((ACC_ROWS // 5, D), lambda i: (i, 0)),
        out_shape=jax.ShapeDtypeStruct((ACC_ROWS, D), jnp.float32),
    )
    return k3(partial[0], partial[1], gs.reshape(2, ACC_ROWS))[:S]


# speedup vs baseline: 5.0120x; 1.1461x over previous
import jax
import jax.numpy as jnp
from jax import lax
from jax.experimental import pallas as pl
from jax.experimental.pallas import tpu as pltpu
from jax.experimental.pallas import tpu_sc as plsc

N = 100000
D = 128
S = 5000
R1 = 800
G1 = N // R1
B = 128
NB_FULL = N // B
TAIL = N - NB_FULL * B
NW = 32
KPW = (NB_FULL + NW - 1) // NW
ACC_ROWS = 5120

_HI = jax.lax.Precision.DEFAULT


def _k1_body(x_ref, W1_ref, b1_ref, w2_ref, b2_ref, Wt_ref, bt_ref,
             h_ref, g_ref, gmax_ref, m_scr):
    i = pl.program_id(0)
    xb = x_ref[...]
    a1 = jnp.maximum(
        jnp.dot(xb, W1_ref[...], preferred_element_type=jnp.float32,
                precision=_HI) + b1_ref[...], 0.0)
    g = jnp.sum(a1 * w2_ref[...], axis=1) + b2_ref[0, 0]
    h_ref[...] = jnp.maximum(
        jnp.dot(xb, Wt_ref[...], preferred_element_type=jnp.float32,
                precision=_HI) + bt_ref[...], 0.0)
    g_ref[0, 0] = g

    @pl.when(i == 0)
    def _():
        m_scr[0] = -3e38

    m_scr[0] = jnp.maximum(m_scr[0], jnp.max(g))
    gmax_ref[...] = jnp.broadcast_to(m_scr[0], (1, 1))


_GDN = lax.GatherDimensionNumbers(
    offset_dims=(), collapsed_slice_dims=(0,), start_index_map=(0,))


def _bcast_lane(v, r):
    return lax.gather(v, jnp.full((16, 1), r, jnp.int32), _GDN,
                      slice_sizes=(1,),
                      mode=lax.GatherScatterMode.PROMISE_IN_BOUNDS)


def _zero_rows(ref, n_rows):
    zero16 = jnp.zeros((16,), jnp.float32)

    def zrow(r, carry):
        for c in range(D // 16):
            ref[r, pl.ds(c * 16, 16)] = zero16
        return carry

    lax.fori_loop(0, n_rows, zrow, 0)


def _k2a_body(h_hbm, g_hbm, i_hbm, gm_hbm, out_hbm,
              gv, iv, hv, ehv, gmv, zv, gt, it, ht, eht, acc):
    cid = lax.axis_index("c")
    sid = lax.axis_index("s")
    wid = sid * 2 + cid

    _zero_rows(zv, 32)
    for k in range(10):
        pltpu.sync_copy(zv, acc.at[pl.ds(sid * 320 + k * 32, 32)])
    plsc.subcore_barrier()

    pltpu.sync_copy(gm_hbm, gmv)
    gmx = gmv[...]

    def scale_rows(n_rows, gref, href, ehref):
        def grp(j, carry):
            ev = jnp.exp(gref[pl.ds(j * 16, 16)] - gmx)
            for r in range(16):
                row = j * 16 + r
                eb = _bcast_lane(ev, r)
                for c in range(8):
                    ehref[row, pl.ds(c * 16, 16)] = (
                        href[row, pl.ds(c * 16, 16)] * eb)
            return carry

        lax.fori_loop(0, n_rows // 16, grp, 0)

    def blk(k, carry):
        b = k * NW + wid

        @pl.when(b < NB_FULL)
        def _():
            off = b * B
            pltpu.sync_copy(g_hbm.at[pl.ds(off, B)], gv)
            pltpu.sync_copy(i_hbm.at[pl.ds(off, B)], iv)
            pltpu.sync_copy(h_hbm.at[pl.ds(off, B)], hv)
            scale_rows(B, gv, hv, ehv)
            pltpu.sync_copy(ehv, acc.at[iv], add=True)

        return carry

    lax.fori_loop(0, KPW, blk, 0)

    @pl.when(wid == NW - 1)
    def _():
        off = NB_FULL * B
        pltpu.sync_copy(g_hbm.at[pl.ds(off, TAIL)], gt)
        pltpu.sync_copy(i_hbm.at[pl.ds(off, TAIL)], it)
        pltpu.sync_copy(h_hbm.at[pl.ds(off, TAIL)], ht)
        scale_rows(TAIL, gt, ht, eht)
        pltpu.sync_copy(eht, acc.at[it], add=True)

    plsc.subcore_barrier()

    pltpu.sync_copy(acc.at[pl.ds(sid * 320, 320)],
                    out_hbm.at[cid, pl.ds(sid * 320, 320)])


def _k2b_body(g_hbm, i_hbm, gm_hbm, gs_hbm,
              gv, iv, evb, gmv, zv, gt, it, evbt, bnc, gsd, acce):
    cid = lax.axis_index("c")
    sid = lax.axis_index("s")
    wid = sid * 2 + cid
    lane0 = lax.iota(jnp.int32, 16) == 0
    lane = lax.iota(jnp.int32, 16)

    _zero_rows(zv, 32)
    for k in range(10):
        pltpu.sync_copy(zv, acce.at[pl.ds(sid * 320 + k * 32, 32)])
    _zero_rows(evb, B)
    _zero_rows(evbt, TAIL)
    plsc.subcore_barrier()

    pltpu.sync_copy(gm_hbm, gmv)
    gmx = gmv[...]

    def e_rows(n_rows, gref, evref):
        def grp(j, carry):
            ev = jnp.exp(gref[pl.ds(j * 16, 16)] - gmx)
            for r in range(16):
                evref[j * 16 + r, pl.ds(0, 16)] = jnp.where(
                    lane0, _bcast_lane(ev, r), 0.0)
            return carry

        lax.fori_loop(0, n_rows // 16, grp, 0)

    def blk(k, carry):
        b = k * NW + wid

        @pl.when(b < NB_FULL)
        def _():
            off = b * B
            pltpu.sync_copy(g_hbm.at[pl.ds(off, B)], gv)
            pltpu.sync_copy(i_hbm.at[pl.ds(off, B)], iv)
            e_rows(B, gv, evb)
            pltpu.sync_copy(evb, acce.at[iv], add=True)

        return carry

    lax.fori_loop(0, KPW, blk, 0)

    @pl.when(wid == NW - 1)
    def _():
        off = NB_FULL * B
        pltpu.sync_copy(g_hbm.at[pl.ds(off, TAIL)], gt)
        pltpu.sync_copy(i_hbm.at[pl.ds(off, TAIL)], it)
        e_rows(TAIL, gt, evbt)
        pltpu.sync_copy(evbt, acce.at[it], add=True)

    plsc.subcore_barrier()

    for q in range(4):
        pltpu.sync_copy(acce.at[pl.ds(sid * 320 + q * 80, 80)], bnc)
        for j in range(5):
            acc16 = jnp.zeros((16,), jnp.float32)
            for kk in range(16):
                v16 = bnc[j * 16 + kk, pl.ds(0, 16)]
                acc16 = jnp.where(lane == kk, _bcast_lane(v16, 0), acc16)
            gsd[pl.ds(q * 80 + j * 16, 16)] = acc16
    pltpu.sync_copy(gsd, gs_hbm.at[pl.ds(cid * ACC_ROWS + sid * 320, 320)])


def _k3_body(p0_ref, p1_ref, gs_ref, o_ref):
    i = pl.program_id(0)
    num = p0_ref[...] + p1_ref[...]
    den = jnp.sum(gs_ref[:, pl.ds(i * (ACC_ROWS // 5), ACC_ROWS // 5)], axis=0)
    o_ref[...] = num / (den[:, None] + 1e-16)


def kernel(x, index, dim_size, W1, b1, W2, b2, Wt, bt):
    k1 = pl.pallas_call(
        _k1_body,
        grid=(G1,),
        in_specs=[
            pl.BlockSpec((R1, D), lambda i: (i, 0)),
            pl.BlockSpec((D, D // 2), lambda i: (0, 0)),
            pl.BlockSpec((1, D // 2), lambda i: (0, 0)),
            pl.BlockSpec((1, D // 2), lambda i: (0, 0)),
            pl.BlockSpec((1, 1), lambda i: (0, 0)),
            pl.BlockSpec((D, D), lambda i: (0, 0)),
            pl.BlockSpec((1, D), lambda i: (0, 0)),
        ],
        out_specs=[
            pl.BlockSpec((R1, D), lambda i: (i, 0)),
            pl.BlockSpec((1, 1, R1), lambda i: (i, 0, 0)),
            pl.BlockSpec((1, 1), lambda i: (0, 0)),
        ],
        out_shape=[
            jax.ShapeDtypeStruct((N, D), jnp.float32),
            jax.ShapeDtypeStruct((G1, 1, R1), jnp.float32),
            jax.ShapeDtypeStruct((1, 1), jnp.float32),
        ],
        scratch_shapes=[pltpu.SMEM((1,), jnp.float32)],
    )
    h, g3, gmax = k1(x, W1, b1.reshape(1, D // 2), W2.reshape(1, D // 2),
                     b2.reshape(1, 1), Wt, bt.reshape(1, D))
    g_flat = g3.reshape(N)
    gmax16 = jnp.broadcast_to(gmax.reshape(1), (16,))

    mesh = plsc.VectorSubcoreMesh(core_axis_name="c", subcore_axis_name="s",
                                  num_cores=2, num_subcores=16)
    k2a = pl.kernel(
        _k2a_body,
        out_type=jax.ShapeDtypeStruct((2, ACC_ROWS, D), jnp.float32),
        mesh=mesh,
        scratch_types=[
            pltpu.VMEM((B,), jnp.float32),
            pltpu.VMEM((B,), jnp.int32),
            pltpu.VMEM((B, D), jnp.float32),
            pltpu.VMEM((B, D), jnp.float32),
            pltpu.VMEM((16,), jnp.float32),
            pltpu.VMEM((32, D), jnp.float32),
            pltpu.VMEM((TAIL,), jnp.float32),
            pltpu.VMEM((TAIL,), jnp.int32),
            pltpu.VMEM((TAIL, D), jnp.float32),
            pltpu.VMEM((TAIL, D), jnp.float32),
            pltpu.VMEM_SHARED((ACC_ROWS, D), jnp.float32),
        ],
    )
    partial = k2a(h, g_flat, index, gmax16)

    k2b = pl.kernel(
        _k2b_body,
        out_type=jax.ShapeDtypeStruct((2 * ACC_ROWS,), jnp.float32),
        mesh=mesh,
        scratch_types=[
            pltpu.VMEM((B,), jnp.float32),
            pltpu.VMEM((B,), jnp.int32),
            pltpu.VMEM((B, D), jnp.float32),
            pltpu.VMEM((16,), jnp.float32),
            pltpu.VMEM((32, D), jnp.float32),
            pltpu.VMEM((TAIL,), jnp.float32),
            pltpu.VMEM((TAIL,), jnp.int32),
            pltpu.VMEM((TAIL, D), jnp.float32),
            pltpu.VMEM((80, D), jnp.float32),
            pltpu.VMEM((320,), jnp.float32),
            pltpu.VMEM_SHARED((ACC_ROWS, D), jnp.float32),
        ],
    )
    gs = k2b(g_flat, index, gmax16)

    k3 = pl.pallas_call(
        _k3_body,
        grid=(5,),
        in_specs=[
            pl.BlockSpec((ACC_ROWS // 5, D), lambda i: (i, 0)),
            pl.BlockSpec((ACC_ROWS // 5, D), lambda i: (i, 0)),
            pl.BlockSpec((2, ACC_ROWS), lambda i: (0, 0)),
        ],
        out_specs=pl.BlockSpec((ACC_ROWS // 5, D), lambda i: (i, 0)),
        out_shape=jax.ShapeDtypeStruct((ACC_ROWS, D), jnp.float32),
    )
    return k3(partial[0], partial[1], gs.reshape(2, ACC_ROWS))[:S]


# speedup vs baseline: 5.9020x; 1.1776x over previous
import jax
import jax.numpy as jnp
from jax import lax
from jax.experimental import pallas as pl
from jax.experimental.pallas import tpu as pltpu
from jax.experimental.pallas import tpu_sc as plsc

N = 100000
D = 128
S = 5000
R1 = 800
G1 = N // R1
B = 128
NB_FULL = N // B
TAIL = N - NB_FULL * B
NW = 32
KPW = (NB_FULL + NW - 1) // NW
ACC_ROWS = 5120

_HI = jax.lax.Precision.DEFAULT


def _k1_body(x_ref, W1_ref, b1_ref, w2_ref, b2_ref, Wt_ref, bt_ref,
             h_ref, g_ref, gmax_ref, m_scr):
    i = pl.program_id(0)
    xb = x_ref[...]
    a1 = jnp.maximum(
        jnp.dot(xb, W1_ref[...], preferred_element_type=jnp.float32,
                precision=_HI) + b1_ref[...], 0.0)
    g = jnp.sum(a1 * w2_ref[...], axis=1) + b2_ref[0, 0]
    h_ref[...] = jnp.maximum(
        jnp.dot(xb, Wt_ref[...], preferred_element_type=jnp.float32,
                precision=_HI) + bt_ref[...], 0.0)
    g_ref[0, 0] = g

    @pl.when(i == 0)
    def _():
        m_scr[0] = -3e38

    m_scr[0] = jnp.maximum(m_scr[0], jnp.max(g))
    gmax_ref[...] = jnp.broadcast_to(m_scr[0], (1, 1))


_GDN = lax.GatherDimensionNumbers(
    offset_dims=(), collapsed_slice_dims=(0,), start_index_map=(0,))


def _bcast_lane(v, r):
    return lax.gather(v, jnp.full((16, 1), r, jnp.int32), _GDN,
                      slice_sizes=(1,),
                      mode=lax.GatherScatterMode.PROMISE_IN_BOUNDS)


def _zero_rows(ref, n_rows):
    zero16 = jnp.zeros((16,), jnp.float32)

    def zrow(r, carry):
        for c in range(D // 16):
            ref[r, pl.ds(c * 16, 16)] = zero16
        return carry

    lax.fori_loop(0, n_rows, zrow, 0)


def _k2a_body(h_hbm, g_hbm, i_hbm, gm_hbm, out_hbm,
              gv0, gv1, iv0, iv1, hv0, hv1, ehv, gmv, zv, gt, it, ht, eht,
              sg0, sg1, si0, si1, sh0, sh1, acc):
    cid = lax.axis_index("c")
    sid = lax.axis_index("s")
    wid = sid * 2 + cid
    gvs, ivs, hvs = (gv0, gv1), (iv0, iv1), (hv0, hv1)
    sgs, sis, shs = (sg0, sg1), (si0, si1), (sh0, sh1)

    _zero_rows(zv, 32)
    for k in range(10):
        pltpu.sync_copy(zv, acc.at[pl.ds(sid * 320 + k * 32, 32)])
    plsc.subcore_barrier()

    pltpu.sync_copy(gm_hbm, gmv)
    gmx = gmv[...]

    def scale_rows(n_rows, gref, href, ehref):
        def grp(j, carry):
            ev = jnp.exp(gref[pl.ds(j * 16, 16)] - gmx)
            for r in range(16):
                row = j * 16 + r
                eb = _bcast_lane(ev, r)
                for c in range(8):
                    ehref[row, pl.ds(c * 16, 16)] = (
                        href[row, pl.ds(c * 16, 16)] * eb)
            return carry

        lax.fori_loop(0, n_rows // 16, grp, 0)

    def start_fetch(p, b):
        off = b * B
        pltpu.async_copy(g_hbm.at[pl.ds(off, B)], gvs[p], sgs[p])
        pltpu.async_copy(i_hbm.at[pl.ds(off, B)], ivs[p], sis[p])
        pltpu.async_copy(h_hbm.at[pl.ds(off, B)], hvs[p], shs[p])

    def wait_fetch(p, b):
        off = b * B
        pltpu.make_async_copy(g_hbm.at[pl.ds(off, B)], gvs[p], sgs[p]).wait()
        pltpu.make_async_copy(i_hbm.at[pl.ds(off, B)], ivs[p], sis[p]).wait()
        pltpu.make_async_copy(h_hbm.at[pl.ds(off, B)], hvs[p], shs[p]).wait()

    start_fetch(0, wid)

    def pair(o, carry):
        for p in range(2):
            q = o * 2 + p
            b = q * NW + wid
            bn = b + NW

            @pl.when(b < NB_FULL)
            def _():
                @pl.when(bn < NB_FULL)
                def _():
                    start_fetch(1 - p, bn)

                wait_fetch(p, b)
                scale_rows(B, gvs[p], hvs[p], ehv)
                pltpu.sync_copy(ehv, acc.at[ivs[p]], add=True)

        return carry

    lax.fori_loop(0, (KPW + 1) // 2, pair, 0)

    @pl.when(wid == NW - 1)
    def _():
        off = NB_FULL * B
        pltpu.sync_copy(g_hbm.at[pl.ds(off, TAIL)], gt)
        pltpu.sync_copy(i_hbm.at[pl.ds(off, TAIL)], it)
        pltpu.sync_copy(h_hbm.at[pl.ds(off, TAIL)], ht)
        scale_rows(TAIL, gt, ht, eht)
        pltpu.sync_copy(eht, acc.at[it], add=True)

    plsc.subcore_barrier()

    pltpu.sync_copy(acc.at[pl.ds(sid * 320, 320)],
                    out_hbm.at[cid, pl.ds(sid * 320, 320)])


def _k2b_body(g_hbm, i_hbm, gm_hbm, gs_hbm,
              gv, iv, evb, gmv, zv, gt, it, evbt, bnc, gsd, acce):
    cid = lax.axis_index("c")
    sid = lax.axis_index("s")
    wid = sid * 2 + cid
    lane0 = lax.iota(jnp.int32, 16) == 0
    lane = lax.iota(jnp.int32, 16)

    _zero_rows(zv, 32)
    for k in range(10):
        pltpu.sync_copy(zv, acce.at[pl.ds(sid * 320 + k * 32, 32)])
    _zero_rows(evb, B)
    _zero_rows(evbt, TAIL)
    plsc.subcore_barrier()

    pltpu.sync_copy(gm_hbm, gmv)
    gmx = gmv[...]

    def e_rows(n_rows, gref, evref):
        def grp(j, carry):
            ev = jnp.exp(gref[pl.ds(j * 16, 16)] - gmx)
            for r in range(16):
                evref[j * 16 + r, pl.ds(0, 16)] = jnp.where(
                    lane0, _bcast_lane(ev, r), 0.0)
            return carry

        lax.fori_loop(0, n_rows // 16, grp, 0)

    def blk(k, carry):
        b = k * NW + wid

        @pl.when(b < NB_FULL)
        def _():
            off = b * B
            pltpu.sync_copy(g_hbm.at[pl.ds(off, B)], gv)
            pltpu.sync_copy(i_hbm.at[pl.ds(off, B)], iv)
            e_rows(B, gv, evb)
            pltpu.sync_copy(evb, acce.at[iv], add=True)

        return carry

    lax.fori_loop(0, KPW, blk, 0)

    @pl.when(wid == NW - 1)
    def _():
        off = NB_FULL * B
        pltpu.sync_copy(g_hbm.at[pl.ds(off, TAIL)], gt)
        pltpu.sync_copy(i_hbm.at[pl.ds(off, TAIL)], it)
        e_rows(TAIL, gt, evbt)
        pltpu.sync_copy(evbt, acce.at[it], add=True)

    plsc.subcore_barrier()

    for q in range(4):
        pltpu.sync_copy(acce.at[pl.ds(sid * 320 + q * 80, 80)], bnc)
        for j in range(5):
            acc16 = jnp.zeros((16,), jnp.float32)
            for kk in range(16):
                v16 = bnc[j * 16 + kk, pl.ds(0, 16)]
                acc16 = jnp.where(lane == kk, _bcast_lane(v16, 0), acc16)
            gsd[pl.ds(q * 80 + j * 16, 16)] = acc16
    pltpu.sync_copy(gsd, gs_hbm.at[pl.ds(cid * ACC_ROWS + sid * 320, 320)])


def _k3_body(p0_ref, p1_ref, gs_ref, o_ref):
    i = pl.program_id(0)
    num = p0_ref[...] + p1_ref[...]
    den = jnp.sum(gs_ref[:, pl.ds(i * (ACC_ROWS // 5), ACC_ROWS // 5)], axis=0)
    o_ref[...] = num / (den[:, None] + 1e-16)


def kernel(x, index, dim_size, W1, b1, W2, b2, Wt, bt):
    k1 = pl.pallas_call(
        _k1_body,
        grid=(G1,),
        in_specs=[
            pl.BlockSpec((R1, D), lambda i: (i, 0)),
            pl.BlockSpec((D, D // 2), lambda i: (0, 0)),
            pl.BlockSpec((1, D // 2), lambda i: (0, 0)),
            pl.BlockSpec((1, D // 2), lambda i: (0, 0)),
            pl.BlockSpec((1, 1), lambda i: (0, 0)),
            pl.BlockSpec((D, D), lambda i: (0, 0)),
            pl.BlockSpec((1, D), lambda i: (0, 0)),
        ],
        out_specs=[
            pl.BlockSpec((R1, D), lambda i: (i, 0)),
            pl.BlockSpec((1, 1, R1), lambda i: (i, 0, 0)),
            pl.BlockSpec((1, 1), lambda i: (0, 0)),
        ],
        out_shape=[
            jax.ShapeDtypeStruct((N, D), jnp.float32),
            jax.ShapeDtypeStruct((G1, 1, R1), jnp.float32),
            jax.ShapeDtypeStruct((1, 1), jnp.float32),
        ],
        scratch_shapes=[pltpu.SMEM((1,), jnp.float32)],
    )
    h, g3, gmax = k1(x, W1, b1.reshape(1, D // 2), W2.reshape(1, D // 2),
                     b2.reshape(1, 1), Wt, bt.reshape(1, D))
    g_flat = g3.reshape(N)
    gmax16 = jnp.broadcast_to(gmax.reshape(1), (16,))

    mesh = plsc.VectorSubcoreMesh(core_axis_name="c", subcore_axis_name="s",
                                  num_cores=2, num_subcores=16)
    k2a = pl.kernel(
        _k2a_body,
        out_type=jax.ShapeDtypeStruct((2, ACC_ROWS, D), jnp.float32),
        mesh=mesh,
        scratch_types=[
            pltpu.VMEM((B,), jnp.float32),
            pltpu.VMEM((B,), jnp.float32),
            pltpu.VMEM((B,), jnp.int32),
            pltpu.VMEM((B,), jnp.int32),
            pltpu.VMEM((B, D), jnp.float32),
            pltpu.VMEM((B, D), jnp.float32),
            pltpu.VMEM((B, D), jnp.float32),
            pltpu.VMEM((16,), jnp.float32),
            pltpu.VMEM((32, D), jnp.float32),
            pltpu.VMEM((TAIL,), jnp.float32),
            pltpu.VMEM((TAIL,), jnp.int32),
            pltpu.VMEM((TAIL, D), jnp.float32),
            pltpu.VMEM((TAIL, D), jnp.float32),
            pltpu.SemaphoreType.DMA,
            pltpu.SemaphoreType.DMA,
            pltpu.SemaphoreType.DMA,
            pltpu.SemaphoreType.DMA,
            pltpu.SemaphoreType.DMA,
            pltpu.SemaphoreType.DMA,
            pltpu.VMEM_SHARED((ACC_ROWS, D), jnp.float32),
        ],
    )
    partial = k2a(h, g_flat, index, gmax16)

    k2b = pl.kernel(
        _k2b_body,
        out_type=jax.ShapeDtypeStruct((2 * ACC_ROWS,), jnp.float32),
        mesh=mesh,
        scratch_types=[
            pltpu.VMEM((B,), jnp.float32),
            pltpu.VMEM((B,), jnp.int32),
            pltpu.VMEM((B, D), jnp.float32),
            pltpu.VMEM((16,), jnp.float32),
            pltpu.VMEM((32, D), jnp.float32),
            pltpu.VMEM((TAIL,), jnp.float32),
            pltpu.VMEM((TAIL,), jnp.int32),
            pltpu.VMEM((TAIL, D), jnp.float32),
            pltpu.VMEM((80, D), jnp.float32),
            pltpu.VMEM((320,), jnp.float32),
            pltpu.VMEM_SHARED((ACC_ROWS, D), jnp.float32),
        ],
    )
    gs = k2b(g_flat, index, gmax16)

    k3 = pl.pallas_call(
        _k3_body,
        grid=(5,),
        in_specs=[
            pl.BlockSpec((ACC_ROWS // 5, D), lambda i: (i, 0)),
            pl.BlockSpec((ACC_ROWS // 5, D), lambda i: (i, 0)),
            pl.BlockSpec((2, ACC_ROWS), lambda i: (0, 0)),
        ],
        out_specs=pl.BlockSpec((ACC_ROWS // 5, D), lambda i: (i, 0)),
        out_shape=jax.ShapeDtypeStruct((ACC_ROWS, D), jnp.float32),
    )
    return k3(partial[0], partial[1], gs.reshape(2, ACC_ROWS))[:S]


# speedup vs baseline: 6.9244x; 1.1732x over previous
import jax
import jax.numpy as jnp
from jax import lax
from jax.experimental import pallas as pl
from jax.experimental.pallas import tpu as pltpu
from jax.experimental.pallas import tpu_sc as plsc

N = 100000
D = 128
S = 5000
R1 = 2000
G1 = N // R1
B = 128
NB_FULL = N // B
TAIL = N - NB_FULL * B
NW = 32
KPW = (NB_FULL + NW - 1) // NW
ACC_ROWS = 5120

_HI = jax.lax.Precision.DEFAULT


def _k1_body(x_ref, W1_ref, b1_ref, w2_ref, b2_ref, Wt_ref, bt_ref,
             h_ref, g_ref, gmax_ref, m_scr):
    i = pl.program_id(0)
    xb = x_ref[...]
    a1 = jnp.maximum(
        jnp.dot(xb, W1_ref[...], preferred_element_type=jnp.float32,
                precision=_HI) + b1_ref[...], 0.0)
    g = jnp.dot(a1, w2_ref[...], preferred_element_type=jnp.float32,
                precision=_HI)[:, 0] + b2_ref[0, 0]
    h_ref[...] = jnp.maximum(
        jnp.dot(xb, Wt_ref[...], preferred_element_type=jnp.float32,
                precision=_HI) + bt_ref[...], 0.0)
    g_ref[0, 0] = g

    @pl.when(i == 0)
    def _():
        m_scr[0] = -3e38

    m_scr[0] = jnp.maximum(m_scr[0], jnp.max(g))
    gmax_ref[...] = jnp.broadcast_to(m_scr[0], (1, 1))


_GDN = lax.GatherDimensionNumbers(
    offset_dims=(), collapsed_slice_dims=(0,), start_index_map=(0,))


def _bcast_lane(v, r):
    return lax.gather(v, jnp.full((16, 1), r, jnp.int32), _GDN,
                      slice_sizes=(1,),
                      mode=lax.GatherScatterMode.PROMISE_IN_BOUNDS)


def _zero_rows(ref, n_rows):
    zero16 = jnp.zeros((16,), jnp.float32)

    def zrow(r, carry):
        for c in range(D // 16):
            ref[r, pl.ds(c * 16, 16)] = zero16
        return carry

    lax.fori_loop(0, n_rows, zrow, 0)


def _k2a_body(h_hbm, g_hbm, i_hbm, gm_hbm, out_hbm,
              gv0, gv1, iv0, iv1, hv0, hv1, ehv, gmv, zv, gt, it, ht, eht,
              sg0, sg1, si0, si1, sh0, sh1, acc):
    cid = lax.axis_index("c")
    sid = lax.axis_index("s")
    wid = sid * 2 + cid
    gvs, ivs, hvs = (gv0, gv1), (iv0, iv1), (hv0, hv1)
    sgs, sis, shs = (sg0, sg1), (si0, si1), (sh0, sh1)

    _zero_rows(zv, 32)
    for k in range(10):
        pltpu.sync_copy(zv, acc.at[pl.ds(sid * 320 + k * 32, 32)])
    plsc.subcore_barrier()

    pltpu.sync_copy(gm_hbm, gmv)
    gmx = gmv[...]

    def scale_rows(n_rows, gref, href, ehref):
        def grp(j, carry):
            ev = jnp.exp(gref[pl.ds(j * 16, 16)] - gmx)
            for r in range(16):
                row = j * 16 + r
                eb = _bcast_lane(ev, r)
                for c in range(8):
                    ehref[row, pl.ds(c * 16, 16)] = (
                        href[row, pl.ds(c * 16, 16)] * eb)
            return carry

        lax.fori_loop(0, n_rows // 16, grp, 0)

    def start_fetch(p, b):
        off = b * B
        pltpu.async_copy(g_hbm.at[pl.ds(off, B)], gvs[p], sgs[p])
        pltpu.async_copy(i_hbm.at[pl.ds(off, B)], ivs[p], sis[p])
        pltpu.async_copy(h_hbm.at[pl.ds(off, B)], hvs[p], shs[p])

    def wait_fetch(p, b):
        off = b * B
        pltpu.make_async_copy(g_hbm.at[pl.ds(off, B)], gvs[p], sgs[p]).wait()
        pltpu.make_async_copy(i_hbm.at[pl.ds(off, B)], ivs[p], sis[p]).wait()
        pltpu.make_async_copy(h_hbm.at[pl.ds(off, B)], hvs[p], shs[p]).wait()

    start_fetch(0, wid)

    def pair(o, carry):
        for p in range(2):
            q = o * 2 + p
            b = q * NW + wid
            bn = b + NW

            @pl.when(b < NB_FULL)
            def _():
                @pl.when(bn < NB_FULL)
                def _():
                    start_fetch(1 - p, bn)

                wait_fetch(p, b)
                scale_rows(B, gvs[p], hvs[p], ehv)
                pltpu.sync_copy(ehv, acc.at[ivs[p]], add=True)

        return carry

    lax.fori_loop(0, (KPW + 1) // 2, pair, 0)

    @pl.when(wid == NW - 1)
    def _():
        off = NB_FULL * B
        pltpu.sync_copy(g_hbm.at[pl.ds(off, TAIL)], gt)
        pltpu.sync_copy(i_hbm.at[pl.ds(off, TAIL)], it)
        pltpu.sync_copy(h_hbm.at[pl.ds(off, TAIL)], ht)
        scale_rows(TAIL, gt, ht, eht)
        pltpu.sync_copy(eht, acc.at[it], add=True)

    plsc.subcore_barrier()

    pltpu.sync_copy(acc.at[pl.ds(sid * 320, 320)],
                    out_hbm.at[cid, pl.ds(sid * 320, 320)])


def _k2b_body(g_hbm, i_hbm, gm_hbm, gs_hbm,
              gv, iv, evb, gmv, zv, gt, it, evbt, bnc, gsd, acce):
    cid = lax.axis_index("c")
    sid = lax.axis_index("s")
    wid = sid * 2 + cid
    lane0 = lax.iota(jnp.int32, 16) == 0
    lane = lax.iota(jnp.int32, 16)

    _zero_rows(zv, 32)
    for k in range(10):
        pltpu.sync_copy(zv, acce.at[pl.ds(sid * 320 + k * 32, 32)])
    _zero_rows(evb, B)
    _zero_rows(evbt, TAIL)
    plsc.subcore_barrier()

    pltpu.sync_copy(gm_hbm, gmv)
    gmx = gmv[...]

    def e_rows(n_rows, gref, evref):
        def grp(j, carry):
            ev = jnp.exp(gref[pl.ds(j * 16, 16)] - gmx)
            for r in range(16):
                evref[j * 16 + r, pl.ds(0, 16)] = jnp.where(
                    lane0, _bcast_lane(ev, r), 0.0)
            return carry

        lax.fori_loop(0, n_rows // 16, grp, 0)

    def blk(k, carry):
        b = k * NW + wid

        @pl.when(b < NB_FULL)
        def _():
            off = b * B
            pltpu.sync_copy(g_hbm.at[pl.ds(off, B)], gv)
            pltpu.sync_copy(i_hbm.at[pl.ds(off, B)], iv)
            e_rows(B, gv, evb)
            pltpu.sync_copy(evb, acce.at[iv], add=True)

        return carry

    lax.fori_loop(0, KPW, blk, 0)

    @pl.when(wid == NW - 1)
    def _():
        off = NB_FULL * B
        pltpu.sync_copy(g_hbm.at[pl.ds(off, TAIL)], gt)
        pltpu.sync_copy(i_hbm.at[pl.ds(off, TAIL)], it)
        e_rows(TAIL, gt, evbt)
        pltpu.sync_copy(evbt, acce.at[it], add=True)

    plsc.subcore_barrier()

    for q in range(4):
        pltpu.sync_copy(acce.at[pl.ds(sid * 320 + q * 80, 80)], bnc)
        for j in range(5):
            acc16 = jnp.zeros((16,), jnp.float32)
            for kk in range(16):
                v16 = bnc[j * 16 + kk, pl.ds(0, 16)]
                acc16 = jnp.where(lane == kk, _bcast_lane(v16, 0), acc16)
            gsd[pl.ds(q * 80 + j * 16, 16)] = acc16
    pltpu.sync_copy(gsd, gs_hbm.at[pl.ds(cid * ACC_ROWS + sid * 320, 320)])


def _k3_body(p0_ref, p1_ref, gs_ref, o_ref):
    i = pl.program_id(0)
    num = p0_ref[...] + p1_ref[...]
    den = jnp.sum(gs_ref[:, pl.ds(i * (ACC_ROWS // 5), ACC_ROWS // 5)], axis=0)
    o_ref[...] = num / (den[:, None] + 1e-16)


def kernel(x, index, dim_size, W1, b1, W2, b2, Wt, bt):
    k1 = pl.pallas_call(
        _k1_body,
        grid=(G1,),
        in_specs=[
            pl.BlockSpec((R1, D), lambda i: (i, 0)),
            pl.BlockSpec((D, D // 2), lambda i: (0, 0)),
            pl.BlockSpec((1, D // 2), lambda i: (0, 0)),
            pl.BlockSpec((D // 2, 8), lambda i: (0, 0)),
            pl.BlockSpec((1, 1), lambda i: (0, 0)),
            pl.BlockSpec((D, D), lambda i: (0, 0)),
            pl.BlockSpec((1, D), lambda i: (0, 0)),
        ],
        out_specs=[
            pl.BlockSpec((R1, D), lambda i: (i, 0)),
            pl.BlockSpec((1, 1, R1), lambda i: (i, 0, 0)),
            pl.BlockSpec((1, 1), lambda i: (0, 0)),
        ],
        out_shape=[
            jax.ShapeDtypeStruct((N, D), jnp.float32),
            jax.ShapeDtypeStruct((G1, 1, R1), jnp.float32),
            jax.ShapeDtypeStruct((1, 1), jnp.float32),
        ],
        scratch_shapes=[pltpu.SMEM((1,), jnp.float32)],
    )
    w2p = jnp.concatenate(
        [W2, jnp.zeros((D // 2, 7), jnp.float32)], axis=1)
    h, g3, gmax = k1(x, W1, b1.reshape(1, D // 2), w2p,
                     b2.reshape(1, 1), Wt, bt.reshape(1, D))
    g_flat = g3.reshape(N)
    gmax16 = jnp.broadcast_to(gmax.reshape(1), (16,))

    mesh = plsc.VectorSubcoreMesh(core_axis_name="c", subcore_axis_name="s",
                                  num_cores=2, num_subcores=16)
    k2a = pl.kernel(
        _k2a_body,
        out_type=jax.ShapeDtypeStruct((2, ACC_ROWS, D), jnp.float32),
        mesh=mesh,
        scratch_types=[
            pltpu.VMEM((B,), jnp.float32),
            pltpu.VMEM((B,), jnp.float32),
            pltpu.VMEM((B,), jnp.int32),
            pltpu.VMEM((B,), jnp.int32),
            pltpu.VMEM((B, D), jnp.float32),
            pltpu.VMEM((B, D), jnp.float32),
            pltpu.VMEM((B, D), jnp.float32),
            pltpu.VMEM((16,), jnp.float32),
            pltpu.VMEM((32, D), jnp.float32),
            pltpu.VMEM((TAIL,), jnp.float32),
            pltpu.VMEM((TAIL,), jnp.int32),
            pltpu.VMEM((TAIL, D), jnp.float32),
            pltpu.VMEM((TAIL, D), jnp.float32),
            pltpu.SemaphoreType.DMA,
            pltpu.SemaphoreType.DMA,
            pltpu.SemaphoreType.DMA,
            pltpu.SemaphoreType.DMA,
            pltpu.SemaphoreType.DMA,
            pltpu.SemaphoreType.DMA,
            pltpu.VMEM_SHARED((ACC_ROWS, D), jnp.float32),
        ],
    )
    partial = k2a(h, g_flat, index, gmax16)

    k2b = pl.kernel(
        _k2b_body,
        out_type=jax.ShapeDtypeStruct((2 * ACC_ROWS,), jnp.float32),
        mesh=mesh,
        scratch_types=[
            pltpu.VMEM((B,), jnp.float32),
            pltpu.VMEM((B,), jnp.int32),
            pltpu.VMEM((B, D), jnp.float32),
            pltpu.VMEM((16,), jnp.float32),
            pltpu.VMEM((32, D), jnp.float32),
            pltpu.VMEM((TAIL,), jnp.float32),
            pltpu.VMEM((TAIL,), jnp.int32),
            pltpu.VMEM((TAIL, D), jnp.float32),
            pltpu.VMEM((80, D), jnp.float32),
            pltpu.VMEM((320,), jnp.float32),
            pltpu.VMEM_SHARED((ACC_ROWS, D), jnp.float32),
        ],
    )
    gs = k2b(g_flat, index, gmax16)

    k3 = pl.pallas_call(
        _k3_body,
        grid=(5,),
        in_specs=[
            pl.BlockSpec((ACC_ROWS // 5, D), lambda i: (i, 0)),
            pl.BlockSpec((ACC_ROWS // 5, D), lambda i: (i, 0)),
            pl.BlockSpec((2, ACC_ROWS), lambda i: (0, 0)),
        ],
        out_specs=pl.BlockSpec((ACC_ROWS // 5, D), lambda i: (i, 0)),
        out_shape=jax.ShapeDtypeStruct((ACC_ROWS, D), jnp.float32),
    )
    return k3(partial[0], partial[1], gs.reshape(2, ACC_ROWS))[:S]
